# baseline (speedup 1.0000x reference)
import functools

import jax
import jax.numpy as jnp
from jax import lax
from jax.experimental import pallas as pl
from jax.experimental.pallas import tpu as pltpu
from jax.experimental.pallas import tpu_sc as plsc

N = 10000
E = 320000
D = 128
HD = D // 2
NC, NS = 2, 16
EPT = E // NS
CH = 128
EPTP = 20480
NCHUNK = EPTP // CH
K = 4
NBATCH = NCHUNK // K
NPAD = 10240
RPT = NPAD // NS
DCH = EPTP * NS // (NC * NS) // CH
BLK = 1000
GRID = N // BLK

_sc_mesh = plsc.VectorSubcoreMesh(core_axis_name="c", subcore_axis_name="s")


@functools.partial(
    pl.kernel,
    out_type=jax.ShapeDtypeStruct((NC * NPAD,), jnp.float32),
    mesh=_sc_mesh,
    scratch_types=[
        pltpu.VMEM((DCH, CH), jnp.int32),
        pltpu.VMEM((CH,), jnp.float32),
        pltpu.VMEM((RPT,), jnp.float32),
        pltpu.VMEM_SHARED((NPAD,), jnp.float32),
    ],
)
def _deg_kernel(col_hbm, out_hbm, cidx, ones_v, zer_v, acc):
    c = lax.axis_index("c")
    s = lax.axis_index("s")
    wid = s * NC + c
    for j in range(CH // 16):
        ones_v[pl.ds(j * 16, 16)] = jnp.ones((16,), jnp.float32)

    def zbody(j, _):
        zer_v[pl.ds(j * 16, 16)] = jnp.zeros((16,), jnp.float32)
        return 0

    lax.fori_loop(0, RPT // 16, zbody, 0)
    pltpu.sync_copy(zer_v, acc.at[pl.ds(s * RPT, RPT)])
    pltpu.sync_copy(col_hbm.at[wid], cidx)
    plsc.subcore_barrier()

    def body(i, _):
        pltpu.sync_copy(ones_v, acc.at[cidx.at[i]], add=True)
        return 0

    lax.fori_loop(0, DCH, body, 0)
    plsc.subcore_barrier()
    pltpu.sync_copy(acc.at[pl.ds(s * RPT, RPT)],
                    out_hbm.at[pl.ds(c * NPAD + s * RPT, RPT)])


@functools.partial(
    pl.kernel,
    out_type=jax.ShapeDtypeStruct((NC * NPAD, HD), jnp.float32),
    mesh=_sc_mesh,
    scratch_types=[
        pltpu.VMEM((4 * K, CH), jnp.int32),
        pltpu.VMEM((4 * K, CH), jnp.int32),
        pltpu.VMEM((2, K * CH, HD), jnp.float32),
        pltpu.VMEM_SHARED((NPAD, HD), jnp.float32),
        pltpu.SemaphoreType.DMA,
        pltpu.SemaphoreType.DMA,
        pltpu.SemaphoreType.DMA,
    ],
    compiler_params=pltpu.CompilerParams(use_tc_tiling_on_sc=False),
)
def _scatter_kernel(ga_hbm, gb_hbm, row_hbm, col_hbm, zros_hbm, out_hbm,
                    ridxb, cidxb, bufs, acc, sem_g, sem_s, sem_i):
    c = lax.axis_index("c")
    s = lax.axis_index("s")

    def fire_i(b):
        bc = jnp.minimum(b, NBATCH - 1)
        slot = (b % 4) * K
        pltpu.async_copy(row_hbm.at[s, pl.ds(bc * K, K)],
                         ridxb.at[pl.ds(slot, K)], sem_i)
        pltpu.async_copy(col_hbm.at[s, pl.ds(bc * K, K)],
                         cidxb.at[pl.ds(slot, K)], sem_i)

    def drain_i():
        pltpu.make_async_copy(row_hbm.at[s, pl.ds(0, K)],
                              ridxb.at[pl.ds(0, K)], sem_i).wait()
        pltpu.make_async_copy(col_hbm.at[s, pl.ds(0, K)],
                              cidxb.at[pl.ds(0, K)], sem_i).wait()

    def run(tbl):
        def fire_g(par, b):
            slot = (b % 4) * K
            for j in range(K):
                pltpu.async_copy(tbl.at[ridxb.at[slot + j]],
                                 bufs.at[par, pl.ds(j * CH, CH)], sem_g)

        def drain_g(par):
            pltpu.make_async_copy(tbl.at[pl.ds(0, K * CH)], bufs.at[par],
                                  sem_g).wait()

        def fire_s(par, b):
            slot = (b % 4) * K
            for j in range(K):
                pltpu.async_copy(bufs.at[par, pl.ds(j * CH, CH)],
                                 acc.at[cidxb.at[slot + j]], sem_s, add=True)

        def drain_s(par):
            for j in range(K):
                pltpu.make_async_copy(bufs.at[par, pl.ds(j * CH, CH)],
                                      acc.at[cidxb.at[0]], sem_s).wait()

        for b in range(4):
            fire_i(b)
        pltpu.sync_copy(zros_hbm, acc.at[pl.ds(s * RPT, RPT)])
        drain_i()
        fire_g(0, 0)
        drain_i()
        fire_g(1, 1)
        plsc.subcore_barrier()

        def body(b, _):
            par = b % 2
            drain_g(par)
            fire_s(par, b)
            drain_s(par)
            fire_i(b + 4)
            drain_i()
            fire_g(par, b + 2)
            return 0

        lax.fori_loop(0, NBATCH - 2, body, 0)
        for b in (NBATCH - 2, NBATCH - 1):
            par = b % 2
            drain_g(par)
            fire_s(par, b)
            drain_s(par)
        drain_i()
        drain_i()

    @pl.when(c == 0)
    def _():
        run(ga_hbm)

    @pl.when(c == 1)
    def _():
        run(gb_hbm)

    plsc.subcore_barrier()
    pltpu.sync_copy(acc.at[pl.ds(s * RPT, RPT)],
                    out_hbm.at[pl.ds(c * NPAD + s * RPT, RPT)])


def _mm1_body(x_ref, w_ref, b_ref, dp_ref, ga_ref, gb_ref, dis_ref):
    h = lax.dot_general(x_ref[...], w_ref[...], (((1,), (1,)), ((), ())),
                        preferred_element_type=jnp.float32) + b_ref[...]
    deg = 1.0 + dp_ref[:, 0:1] + dp_ref[:, 1:2]
    dis = lax.rsqrt(deg)
    dis_ref[...] = dis
    g = dis * h
    ga_ref[...] = g[:, :HD]
    gb_ref[...] = g[:, HD:]


_mm1 = pl.pallas_call(
    _mm1_body,
    grid=(GRID,),
    in_specs=[
        pl.BlockSpec((BLK, D), lambda i: (i, 0)),
        pl.BlockSpec((D, D), lambda i: (0, 0)),
        pl.BlockSpec((1, D), lambda i: (0, 0)),
        pl.BlockSpec((BLK, 2), lambda i: (i, 0)),
    ],
    out_specs=[
        pl.BlockSpec((BLK, HD), lambda i: (i, 0)),
        pl.BlockSpec((BLK, HD), lambda i: (i, 0)),
        pl.BlockSpec((BLK, 1), lambda i: (i, 0)),
    ],
    out_shape=[
        jax.ShapeDtypeStruct((N, HD), jnp.float32),
        jax.ShapeDtypeStruct((N, HD), jnp.float32),
        jax.ShapeDtypeStruct((N, 1), jnp.float32),
    ],
)


def _mm2_body(ga_ref, gb_ref, pa_ref, pb_ref, dis_ref, w_ref, b_ref,
              g2a_ref, g2b_ref):
    dis = dis_ref[...]
    ha = jnp.maximum(dis * (ga_ref[...] + pa_ref[0]), 0.0)
    hb = jnp.maximum(dis * (gb_ref[...] + pb_ref[0]), 0.0)
    mm = lax.dot_general(ha, w_ref[:, :HD], (((1,), (1,)), ((), ())),
                         preferred_element_type=jnp.float32)
    mm = mm + lax.dot_general(hb, w_ref[:, HD:], (((1,), (1,)), ((), ())),
                              preferred_element_type=jnp.float32)
    g2 = dis * (mm + b_ref[...])
    g2a_ref[...] = g2[:, :HD]
    g2b_ref[...] = g2[:, HD:]


_mm2 = pl.pallas_call(
    _mm2_body,
    grid=(GRID,),
    in_specs=[
        pl.BlockSpec((BLK, HD), lambda i: (i, 0)),
        pl.BlockSpec((BLK, HD), lambda i: (i, 0)),
        pl.BlockSpec((1, BLK, HD), lambda i: (0, i, 0)),
        pl.BlockSpec((1, BLK, HD), lambda i: (1, i, 0)),
        pl.BlockSpec((BLK, 1), lambda i: (i, 0)),
        pl.BlockSpec((D, D), lambda i: (0, 0)),
        pl.BlockSpec((1, D), lambda i: (0, 0)),
    ],
    out_specs=[
        pl.BlockSpec((BLK, HD), lambda i: (i, 0)),
        pl.BlockSpec((BLK, HD), lambda i: (i, 0)),
    ],
    out_shape=[
        jax.ShapeDtypeStruct((N, HD), jnp.float32),
        jax.ShapeDtypeStruct((N, HD), jnp.float32),
    ],
)


def _fin_body(g2a_ref, g2b_ref, pa_ref, pb_ref, dis_ref, o_ref):
    dis = dis_ref[...]
    o_ref[:, :HD] = dis * (g2a_ref[...] + pa_ref[0])
    o_ref[:, HD:] = dis * (g2b_ref[...] + pb_ref[0])


_fin = pl.pallas_call(
    _fin_body,
    grid=(GRID,),
    in_specs=[
        pl.BlockSpec((BLK, HD), lambda i: (i, 0)),
        pl.BlockSpec((BLK, HD), lambda i: (i, 0)),
        pl.BlockSpec((1, BLK, HD), lambda i: (0, i, 0)),
        pl.BlockSpec((1, BLK, HD), lambda i: (1, i, 0)),
        pl.BlockSpec((BLK, 1), lambda i: (i, 0)),
    ],
    out_specs=pl.BlockSpec((BLK, D), lambda i: (i, 0)),
    out_shape=jax.ShapeDtypeStruct((N, D), jnp.float32),
)


def kernel(x, edge_index, W1, b1, W2, b2):
    ei = edge_index.astype(jnp.int32)
    rowp = jnp.pad(ei[0].reshape(NS, EPT), ((0, 0), (0, EPTP - EPT)),
                   constant_values=0).reshape(NS, NCHUNK, CH)
    colp = jnp.pad(ei[1].reshape(NS, EPT), ((0, 0), (0, EPTP - EPT)),
                   constant_values=N).reshape(NS, NCHUNK, CH)
    colw = colp.reshape(NC * NS, DCH, CH)
    zros = jnp.zeros((RPT, HD), jnp.float32)
    degp = _deg_kernel(colw)
    degT = jnp.transpose(degp.reshape(NC, NPAD))
    b1r = b1.reshape(1, D)
    b2r = b2.reshape(1, D)
    g1a, g1b, dis = _mm1(x, W1, b1r, degT)
    p1 = _scatter_kernel(g1a, g1b, rowp, colp, zros).reshape(NC, NPAD, HD)
    g2a, g2b = _mm2(g1a, g1b, p1, p1, dis, W2, b2r)
    p2 = _scatter_kernel(g2a, g2b, rowp, colp, zros).reshape(NC, NPAD, HD)
    return _fin(g2a, g2b, p2, p2, dis)


# speedup vs baseline: 1.4697x; 1.4697x over previous
import functools

import jax
import jax.numpy as jnp
from jax import lax
from jax.experimental import pallas as pl
from jax.experimental.pallas import tpu as pltpu
from jax.experimental.pallas import tpu_sc as plsc

N = 10000
E = 320000
D = 128
HD = D // 2
NC, NS = 2, 16
EPT = E // NS
CH = 80
EPTP = 20160
NCHUNK = EPTP // CH
K = 4
NBATCH = NCHUNK // K
NPAD = 10240
RPT = NPAD // NS
DCH = EPTP * NS // (NC * NS) // CH
BLK = 1000
GRID = N // BLK

_sc_mesh = plsc.VectorSubcoreMesh(core_axis_name="c", subcore_axis_name="s")


@functools.partial(
    pl.kernel,
    out_type=jax.ShapeDtypeStruct((NC * NPAD,), jnp.float32),
    mesh=_sc_mesh,
    scratch_types=[
        pltpu.VMEM((DCH, CH), jnp.int32),
        pltpu.VMEM((CH,), jnp.float32),
        pltpu.VMEM((RPT,), jnp.float32),
        pltpu.VMEM_SHARED((NPAD,), jnp.float32),
    ],
)
def _deg_kernel(col_hbm, out_hbm, cidx, ones_v, zer_v, acc):
    c = lax.axis_index("c")
    s = lax.axis_index("s")
    wid = s * NC + c
    for j in range(CH // 16):
        ones_v[pl.ds(j * 16, 16)] = jnp.ones((16,), jnp.float32)

    def zbody(j, _):
        zer_v[pl.ds(j * 16, 16)] = jnp.zeros((16,), jnp.float32)
        return 0

    lax.fori_loop(0, RPT // 16, zbody, 0)
    pltpu.sync_copy(zer_v, acc.at[pl.ds(s * RPT, RPT)])
    pltpu.sync_copy(col_hbm.at[wid], cidx)
    plsc.subcore_barrier()

    def body(i, _):
        pltpu.sync_copy(ones_v, acc.at[cidx.at[i]], add=True)
        return 0

    lax.fori_loop(0, DCH, body, 0)
    plsc.subcore_barrier()
    pltpu.sync_copy(acc.at[pl.ds(s * RPT, RPT)],
                    out_hbm.at[pl.ds(c * NPAD + s * RPT, RPT)])


@functools.partial(
    pl.kernel,
    out_type=jax.ShapeDtypeStruct((NC * NPAD, HD), jnp.float32),
    mesh=_sc_mesh,
    scratch_types=[
        pltpu.VMEM((4 * K, CH), jnp.int32),
        pltpu.VMEM((4 * K, CH), jnp.int32),
        pltpu.VMEM((2, K * CH, HD), jnp.float32),
        pltpu.VMEM_SHARED((NPAD, HD), jnp.float32),
        pltpu.SemaphoreType.DMA,
        pltpu.SemaphoreType.DMA,
        pltpu.SemaphoreType.DMA,
    ],
    compiler_params=pltpu.CompilerParams(use_tc_tiling_on_sc=False),
)
def _scatter_kernel(ga_hbm, gb_hbm, row_hbm, col_hbm, zros_hbm, out_hbm,
                    ridxb, cidxb, bufs, acc, sem_g, sem_s, sem_i):
    c = lax.axis_index("c")
    s = lax.axis_index("s")

    def fire_i(b):
        bc = jnp.minimum(b, NBATCH - 1)
        slot = (b % 4) * K
        pltpu.async_copy(row_hbm.at[s, pl.ds(bc * K, K)],
                         ridxb.at[pl.ds(slot, K)], sem_i)
        pltpu.async_copy(col_hbm.at[s, pl.ds(bc * K, K)],
                         cidxb.at[pl.ds(slot, K)], sem_i)

    def drain_i():
        pltpu.make_async_copy(row_hbm.at[s, pl.ds(0, K)],
                              ridxb.at[pl.ds(0, K)], sem_i).wait()
        pltpu.make_async_copy(col_hbm.at[s, pl.ds(0, K)],
                              cidxb.at[pl.ds(0, K)], sem_i).wait()

    def run(tbl):
        def fire_g(par, b):
            slot = (b % 4) * K
            for j in range(K):
                pltpu.async_copy(tbl.at[ridxb.at[slot + j]],
                                 bufs.at[par, pl.ds(j * CH, CH)], sem_g)

        def drain_g(par):
            pltpu.make_async_copy(tbl.at[pl.ds(0, K * CH)], bufs.at[par],
                                  sem_g).wait()

        def fire_s(par, b):
            slot = (b % 4) * K
            for j in range(K):
                pltpu.async_copy(bufs.at[par, pl.ds(j * CH, CH)],
                                 acc.at[cidxb.at[slot + j]], sem_s, add=True)

        def drain_s(par):
            for j in range(K):
                pltpu.make_async_copy(bufs.at[par, pl.ds(j * CH, CH)],
                                      acc.at[cidxb.at[0]], sem_s).wait()

        for b in range(4):
            fire_i(b)
        pltpu.sync_copy(zros_hbm, acc.at[pl.ds(s * RPT, RPT)])
        drain_i()
        fire_g(0, 0)
        drain_i()
        fire_g(1, 1)
        plsc.subcore_barrier()

        def body(b, _):
            par = b % 2
            drain_g(par)
            fire_s(par, b)
            drain_s(par)
            fire_i(b + 4)
            drain_i()
            fire_g(par, b + 2)
            return 0

        lax.fori_loop(0, NBATCH - 2, body, 0)
        for b in (NBATCH - 2, NBATCH - 1):
            par = b % 2
            drain_g(par)
            fire_s(par, b)
            drain_s(par)
        drain_i()
        drain_i()

    @pl.when(c == 0)
    def _():
        run(ga_hbm)

    @pl.when(c == 1)
    def _():
        run(gb_hbm)

    plsc.subcore_barrier()
    pltpu.sync_copy(acc.at[pl.ds(s * RPT, RPT)],
                    out_hbm.at[pl.ds(c * NPAD + s * RPT, RPT)])


def _mm1_body(x_ref, w_ref, b_ref, dp_ref, ga_ref, gb_ref, dis_ref):
    h = lax.dot_general(x_ref[...], w_ref[...], (((1,), (1,)), ((), ())),
                        preferred_element_type=jnp.float32) + b_ref[...]
    deg = 1.0 + dp_ref[:, 0:1] + dp_ref[:, 1:2]
    dis = lax.rsqrt(deg)
    dis_ref[...] = dis
    g = dis * h
    ga_ref[...] = g[:, :HD]
    gb_ref[...] = g[:, HD:]


_mm1 = pl.pallas_call(
    _mm1_body,
    grid=(GRID,),
    in_specs=[
        pl.BlockSpec((BLK, D), lambda i: (i, 0)),
        pl.BlockSpec((D, D), lambda i: (0, 0)),
        pl.BlockSpec((1, D), lambda i: (0, 0)),
        pl.BlockSpec((BLK, 2), lambda i: (i, 0)),
    ],
    out_specs=[
        pl.BlockSpec((BLK, HD), lambda i: (i, 0)),
        pl.BlockSpec((BLK, HD), lambda i: (i, 0)),
        pl.BlockSpec((BLK, 1), lambda i: (i, 0)),
    ],
    out_shape=[
        jax.ShapeDtypeStruct((N, HD), jnp.float32),
        jax.ShapeDtypeStruct((N, HD), jnp.float32),
        jax.ShapeDtypeStruct((N, 1), jnp.float32),
    ],
)


def _mm2_body(ga_ref, gb_ref, pa_ref, pb_ref, dis_ref, w_ref, b_ref,
              g2a_ref, g2b_ref):
    dis = dis_ref[...]
    ha = jnp.maximum(dis * (ga_ref[...] + pa_ref[0]), 0.0)
    hb = jnp.maximum(dis * (gb_ref[...] + pb_ref[0]), 0.0)
    mm = lax.dot_general(ha, w_ref[:, :HD], (((1,), (1,)), ((), ())),
                         preferred_element_type=jnp.float32)
    mm = mm + lax.dot_general(hb, w_ref[:, HD:], (((1,), (1,)), ((), ())),
                              preferred_element_type=jnp.float32)
    g2 = dis * (mm + b_ref[...])
    g2a_ref[...] = g2[:, :HD]
    g2b_ref[...] = g2[:, HD:]


_mm2 = pl.pallas_call(
    _mm2_body,
    grid=(GRID,),
    in_specs=[
        pl.BlockSpec((BLK, HD), lambda i: (i, 0)),
        pl.BlockSpec((BLK, HD), lambda i: (i, 0)),
        pl.BlockSpec((1, BLK, HD), lambda i: (0, i, 0)),
        pl.BlockSpec((1, BLK, HD), lambda i: (1, i, 0)),
        pl.BlockSpec((BLK, 1), lambda i: (i, 0)),
        pl.BlockSpec((D, D), lambda i: (0, 0)),
        pl.BlockSpec((1, D), lambda i: (0, 0)),
    ],
    out_specs=[
        pl.BlockSpec((BLK, HD), lambda i: (i, 0)),
        pl.BlockSpec((BLK, HD), lambda i: (i, 0)),
    ],
    out_shape=[
        jax.ShapeDtypeStruct((N, HD), jnp.float32),
        jax.ShapeDtypeStruct((N, HD), jnp.float32),
    ],
)


def _fin_body(g2a_ref, g2b_ref, pa_ref, pb_ref, dis_ref, o_ref):
    dis = dis_ref[...]
    o_ref[:, :HD] = dis * (g2a_ref[...] + pa_ref[0])
    o_ref[:, HD:] = dis * (g2b_ref[...] + pb_ref[0])


_fin = pl.pallas_call(
    _fin_body,
    grid=(GRID,),
    in_specs=[
        pl.BlockSpec((BLK, HD), lambda i: (i, 0)),
        pl.BlockSpec((BLK, HD), lambda i: (i, 0)),
        pl.BlockSpec((1, BLK, HD), lambda i: (0, i, 0)),
        pl.BlockSpec((1, BLK, HD), lambda i: (1, i, 0)),
        pl.BlockSpec((BLK, 1), lambda i: (i, 0)),
    ],
    out_specs=pl.BlockSpec((BLK, D), lambda i: (i, 0)),
    out_shape=jax.ShapeDtypeStruct((N, D), jnp.float32),
)


def kernel(x, edge_index, W1, b1, W2, b2):
    ei = edge_index.astype(jnp.int32)
    rowp = jnp.pad(ei[0].reshape(NS, EPT), ((0, 0), (0, EPTP - EPT)),
                   constant_values=0).reshape(NS, NCHUNK, CH)
    colp = jnp.pad(ei[1].reshape(NS, EPT), ((0, 0), (0, EPTP - EPT)),
                   constant_values=N).reshape(NS, NCHUNK, CH)
    colw = colp.reshape(NC * NS, DCH, CH)
    zros = jnp.zeros((RPT, HD), jnp.float32)
    degp = _deg_kernel(colw)
    degT = jnp.transpose(degp.reshape(NC, NPAD))
    b1r = b1.reshape(1, D)
    b2r = b2.reshape(1, D)
    g1a, g1b, dis = _mm1(x, W1, b1r, degT)
    p1 = _scatter_kernel(g1a, g1b, rowp, colp, zros).reshape(NC, NPAD, HD)
    g2a, g2b = _mm2(g1a, g1b, p1, p1, dis, W2, b2r)
    p2 = _scatter_kernel(g2a, g2b, rowp, colp, zros).reshape(NC, NPAD, HD)
    return _fin(g2a, g2b, p2, p2, dis)


# speedup vs baseline: 1.8857x; 1.2830x over previous
import functools

import jax
import jax.numpy as jnp
from jax import lax
from jax.experimental import pallas as pl
from jax.experimental.pallas import tpu as pltpu
from jax.experimental.pallas import tpu_sc as plsc

N = 10000
E = 320000
D = 128
HD = D // 2
NC, NS = 2, 16
EPT = E // NS
CH = 80
NCHUNK = EPT // CH
K = 4
NBATCH = NCHUNK // K
NPAD = 10240
RPT = NPAD // NS
ZR = 80
EPW = E // (NC * NS)
DCH = EPW // CH
BLK = 1000
GRID = N // BLK

_sc_mesh = plsc.VectorSubcoreMesh(core_axis_name="c", subcore_axis_name="s")


@functools.partial(
    pl.kernel,
    out_type=jax.ShapeDtypeStruct((NC * NPAD,), jnp.float32),
    mesh=_sc_mesh,
    scratch_types=[
        pltpu.VMEM((DCH, CH), jnp.int32),
        pltpu.VMEM((CH,), jnp.float32),
        pltpu.VMEM((RPT,), jnp.float32),
        pltpu.VMEM_SHARED((NPAD,), jnp.float32),
    ],
)
def _deg_kernel(col_hbm, out_hbm, cidx, ones_v, zer_v, acc):
    c = lax.axis_index("c")
    s = lax.axis_index("s")
    wid = s * NC + c
    for j in range(CH // 16):
        ones_v[pl.ds(j * 16, 16)] = jnp.ones((16,), jnp.float32)

    def zbody(j, _):
        zer_v[pl.ds(j * 16, 16)] = jnp.zeros((16,), jnp.float32)
        return 0

    lax.fori_loop(0, RPT // 16, zbody, 0)
    pltpu.sync_copy(zer_v, acc.at[pl.ds(s * RPT, RPT)])
    pltpu.sync_copy(col_hbm.at[wid], cidx)
    plsc.subcore_barrier()

    def body(i, _):
        pltpu.sync_copy(ones_v, acc.at[cidx.at[i]], add=True)
        return 0

    lax.fori_loop(0, DCH, body, 0)
    plsc.subcore_barrier()
    pltpu.sync_copy(acc.at[pl.ds(s * RPT, RPT)],
                    out_hbm.at[pl.ds(c * NPAD + s * RPT, RPT)])


@functools.partial(
    pl.kernel,
    out_type=jax.ShapeDtypeStruct((NC * NPAD, HD), jnp.float32),
    mesh=_sc_mesh,
    scratch_types=[
        pltpu.VMEM((NCHUNK, CH), jnp.int32),
        pltpu.VMEM((NCHUNK, CH), jnp.int32),
        pltpu.VMEM((2 * K, CH, HD), jnp.float32),
        pltpu.VMEM((ZR, HD), jnp.float32),
        pltpu.VMEM_SHARED((NPAD, HD), jnp.float32),
        pltpu.SemaphoreType.DMA,
        pltpu.SemaphoreType.DMA,
    ],
    compiler_params=pltpu.CompilerParams(use_tc_tiling_on_sc=False),
)
def _scatter_kernel(ga_hbm, gb_hbm, row_hbm, col_hbm, out_hbm, ridx, cidx,
                    bufs, zer_v, acc, sem_g, sem_s):
    c = lax.axis_index("c")
    s = lax.axis_index("s")

    def zfill(i, _):
        for j in range(HD // 16):
            zer_v[i, pl.ds(j * 16, 16)] = jnp.zeros((16,), jnp.float32)
        return 0

    lax.fori_loop(0, ZR, zfill, 0)

    def zcopy(k, _):
        pltpu.sync_copy(zer_v, acc.at[pl.ds(s * RPT + k * ZR, ZR)])
        return 0

    lax.fori_loop(0, RPT // ZR, zcopy, 0)
    pltpu.sync_copy(row_hbm.at[s], ridx)
    pltpu.sync_copy(col_hbm.at[s], cidx)
    plsc.subcore_barrier()

    def run(tbl):
        def fire_g(off, b):
            for j in range(K):
                pltpu.async_copy(tbl.at[ridx.at[b * K + j]], bufs.at[off + j],
                                 sem_g)

        def drain_g(off):
            for j in range(K):
                pltpu.make_async_copy(tbl.at[ridx.at[0]], bufs.at[off + j],
                                      sem_g).wait()

        def fire_s(off, b):
            for j in range(K):
                pltpu.async_copy(bufs.at[off + j], acc.at[cidx.at[b * K + j]],
                                 sem_s, add=True)

        def drain_s(off):
            for j in range(K):
                pltpu.make_async_copy(bufs.at[off + j], acc.at[cidx.at[0]],
                                      sem_s).wait()

        fire_g(0, 0)
        fire_g(K, 1)

        def body(b, _):
            par = (b % 2) * K
            drain_g(par)
            fire_s(par, b)
            drain_s(par)
            fire_g(par, b + 2)
            return 0

        lax.fori_loop(0, NBATCH - 2, body, 0)
        for b in (NBATCH - 2, NBATCH - 1):
            par = (b % 2) * K
            drain_g(par)
            fire_s(par, b)
            drain_s(par)
        for e in range(NBATCH * K, NCHUNK):
            pltpu.async_copy(tbl.at[ridx.at[e]], bufs.at[0], sem_g).wait()
            pltpu.sync_copy(bufs.at[0], acc.at[cidx.at[e]], add=True)

    @pl.when(c == 0)
    def _():
        run(ga_hbm)

    @pl.when(c == 1)
    def _():
        run(gb_hbm)

    plsc.subcore_barrier()
    pltpu.sync_copy(acc.at[pl.ds(s * RPT, RPT)],
                    out_hbm.at[pl.ds(c * NPAD + s * RPT, RPT)])


def _mm1_body(x_ref, w_ref, b_ref, dp_ref, ga_ref, gb_ref, dis_ref):
    h = lax.dot_general(x_ref[...], w_ref[...], (((1,), (1,)), ((), ())),
                        preferred_element_type=jnp.float32) + b_ref[...]
    deg = 1.0 + dp_ref[:, 0:1] + dp_ref[:, 1:2]
    dis = lax.rsqrt(deg)
    dis_ref[...] = dis
    g = dis * h
    ga_ref[...] = g[:, :HD]
    gb_ref[...] = g[:, HD:]


_mm1 = pl.pallas_call(
    _mm1_body,
    grid=(GRID,),
    in_specs=[
        pl.BlockSpec((BLK, D), lambda i: (i, 0)),
        pl.BlockSpec((D, D), lambda i: (0, 0)),
        pl.BlockSpec((1, D), lambda i: (0, 0)),
        pl.BlockSpec((BLK, 2), lambda i: (i, 0)),
    ],
    out_specs=[
        pl.BlockSpec((BLK, HD), lambda i: (i, 0)),
        pl.BlockSpec((BLK, HD), lambda i: (i, 0)),
        pl.BlockSpec((BLK, 1), lambda i: (i, 0)),
    ],
    out_shape=[
        jax.ShapeDtypeStruct((N, HD), jnp.float32),
        jax.ShapeDtypeStruct((N, HD), jnp.float32),
        jax.ShapeDtypeStruct((N, 1), jnp.float32),
    ],
)


def _mm2_body(ga_ref, gb_ref, pa_ref, pb_ref, dis_ref, w_ref, b_ref,
              g2a_ref, g2b_ref):
    dis = dis_ref[...]
    ha = jnp.maximum(dis * (ga_ref[...] + pa_ref[0]), 0.0)
    hb = jnp.maximum(dis * (gb_ref[...] + pb_ref[0]), 0.0)
    mm = lax.dot_general(ha, w_ref[:, :HD], (((1,), (1,)), ((), ())),
                         preferred_element_type=jnp.float32)
    mm = mm + lax.dot_general(hb, w_ref[:, HD:], (((1,), (1,)), ((), ())),
                              preferred_element_type=jnp.float32)
    g2 = dis * (mm + b_ref[...])
    g2a_ref[...] = g2[:, :HD]
    g2b_ref[...] = g2[:, HD:]


_mm2 = pl.pallas_call(
    _mm2_body,
    grid=(GRID,),
    in_specs=[
        pl.BlockSpec((BLK, HD), lambda i: (i, 0)),
        pl.BlockSpec((BLK, HD), lambda i: (i, 0)),
        pl.BlockSpec((1, BLK, HD), lambda i: (0, i, 0)),
        pl.BlockSpec((1, BLK, HD), lambda i: (1, i, 0)),
        pl.BlockSpec((BLK, 1), lambda i: (i, 0)),
        pl.BlockSpec((D, D), lambda i: (0, 0)),
        pl.BlockSpec((1, D), lambda i: (0, 0)),
    ],
    out_specs=[
        pl.BlockSpec((BLK, HD), lambda i: (i, 0)),
        pl.BlockSpec((BLK, HD), lambda i: (i, 0)),
    ],
    out_shape=[
        jax.ShapeDtypeStruct((N, HD), jnp.float32),
        jax.ShapeDtypeStruct((N, HD), jnp.float32),
    ],
)


def _fin_body(g2a_ref, g2b_ref, pa_ref, pb_ref, dis_ref, o_ref):
    dis = dis_ref[...]
    o_ref[:, :HD] = dis * (g2a_ref[...] + pa_ref[0])
    o_ref[:, HD:] = dis * (g2b_ref[...] + pb_ref[0])


_fin = pl.pallas_call(
    _fin_body,
    grid=(GRID,),
    in_specs=[
        pl.BlockSpec((BLK, HD), lambda i: (i, 0)),
        pl.BlockSpec((BLK, HD), lambda i: (i, 0)),
        pl.BlockSpec((1, BLK, HD), lambda i: (0, i, 0)),
        pl.BlockSpec((1, BLK, HD), lambda i: (1, i, 0)),
        pl.BlockSpec((BLK, 1), lambda i: (i, 0)),
    ],
    out_specs=pl.BlockSpec((BLK, D), lambda i: (i, 0)),
    out_shape=jax.ShapeDtypeStruct((N, D), jnp.float32),
)


def kernel(x, edge_index, W1, b1, W2, b2):
    ei = edge_index.astype(jnp.int32)
    rowd = ei[0].reshape(NS, NCHUNK, CH)
    cold = ei[1].reshape(NS, NCHUNK, CH)
    colw = ei[1].reshape(NC * NS, DCH, CH)
    degp = _deg_kernel(colw)
    degT = jnp.transpose(degp.reshape(NC, NPAD))
    b1r = b1.reshape(1, D)
    b2r = b2.reshape(1, D)
    g1a, g1b, dis = _mm1(x, W1, b1r, degT)
    p1 = _scatter_kernel(g1a, g1b, rowd, cold).reshape(NC, NPAD, HD)
    g2a, g2b = _mm2(g1a, g1b, p1, p1, dis, W2, b2r)
    p2 = _scatter_kernel(g2a, g2b, rowd, cold).reshape(NC, NPAD, HD)
    return _fin(g2a, g2b, p2, p2, dis)


# speedup vs baseline: 1.9285x; 1.0227x over previous
import functools

import jax
import jax.numpy as jnp
from jax import lax
from jax.experimental import pallas as pl
from jax.experimental.pallas import tpu as pltpu
from jax.experimental.pallas import tpu_sc as plsc

N = 10000
E = 320000
D = 128
HD = D // 2
NC, NS = 2, 16
EPT = E // NS
CH = 80
NCHUNK = EPT // CH
K = 4
NBATCH = NCHUNK // K
NPAD = 10240
RPT = NPAD // NS
ZR = 80
EPW = E // (NC * NS)
DCH = EPW // CH
BLK = 1000
GRID = N // BLK

_sc_mesh = plsc.VectorSubcoreMesh(core_axis_name="c", subcore_axis_name="s")


@functools.partial(
    pl.kernel,
    out_type=jax.ShapeDtypeStruct((NC * NPAD,), jnp.float32),
    mesh=_sc_mesh,
    scratch_types=[
        pltpu.VMEM((DCH, CH), jnp.int32),
        pltpu.VMEM((CH,), jnp.float32),
        pltpu.VMEM((RPT,), jnp.float32),
        pltpu.VMEM_SHARED((NPAD,), jnp.float32),
        pltpu.SemaphoreType.DMA,
    ],
)
def _deg_kernel(col_hbm, out_hbm, cidx, ones_v, zer_v, acc, sem):
    c = lax.axis_index("c")
    s = lax.axis_index("s")
    wid = s * NC + c
    for j in range(CH // 16):
        ones_v[pl.ds(j * 16, 16)] = jnp.ones((16,), jnp.float32)

    def zbody(j, _):
        zer_v[pl.ds(j * 16, 16)] = jnp.zeros((16,), jnp.float32)
        return 0

    lax.fori_loop(0, RPT // 16, zbody, 0)
    pltpu.sync_copy(zer_v, acc.at[pl.ds(s * RPT, RPT)])
    pltpu.sync_copy(col_hbm.at[wid], cidx)
    plsc.subcore_barrier()

    def body(i, _):
        pltpu.async_copy(ones_v, acc.at[cidx.at[i]], sem, add=True)
        return 0

    lax.fori_loop(0, DCH, body, 0)

    def drain(i, _):
        pltpu.make_async_copy(ones_v, acc.at[cidx.at[0]], sem).wait()
        return 0

    lax.fori_loop(0, DCH, drain, 0)
    plsc.subcore_barrier()
    pltpu.sync_copy(acc.at[pl.ds(s * RPT, RPT)],
                    out_hbm.at[pl.ds(c * NPAD + s * RPT, RPT)])


@functools.partial(
    pl.kernel,
    out_type=jax.ShapeDtypeStruct((NC * NPAD, HD), jnp.float32),
    mesh=_sc_mesh,
    scratch_types=[
        pltpu.VMEM((NCHUNK, CH), jnp.int32),
        pltpu.VMEM((NCHUNK, CH), jnp.int32),
        pltpu.VMEM((2 * K, CH, HD), jnp.float32),
        pltpu.VMEM((ZR, HD), jnp.float32),
        pltpu.VMEM_SHARED((NPAD, HD), jnp.float32),
        pltpu.SemaphoreType.DMA,
        pltpu.SemaphoreType.DMA,
    ],
    compiler_params=pltpu.CompilerParams(use_tc_tiling_on_sc=False),
)
def _scatter_kernel(ga_hbm, gb_hbm, row_hbm, col_hbm, out_hbm, ridx, cidx,
                    bufs, zer_v, acc, sem_g, sem_s):
    c = lax.axis_index("c")
    s = lax.axis_index("s")

    def zfill(i, _):
        for j in range(HD // 16):
            zer_v[i, pl.ds(j * 16, 16)] = jnp.zeros((16,), jnp.float32)
        return 0

    lax.fori_loop(0, ZR, zfill, 0)

    def zcopy(k, _):
        pltpu.sync_copy(zer_v, acc.at[pl.ds(s * RPT + k * ZR, ZR)])
        return 0

    lax.fori_loop(0, RPT // ZR, zcopy, 0)
    pltpu.sync_copy(row_hbm.at[s], ridx)
    pltpu.sync_copy(col_hbm.at[s], cidx)
    plsc.subcore_barrier()

    def run(tbl):
        def fire_g(off, b):
            for j in range(K):
                pltpu.async_copy(tbl.at[ridx.at[b * K + j]], bufs.at[off + j],
                                 sem_g)

        def drain_g(off):
            for j in range(K):
                pltpu.make_async_copy(tbl.at[ridx.at[0]], bufs.at[off + j],
                                      sem_g).wait()

        def fire_s(off, b):
            for j in range(K):
                pltpu.async_copy(bufs.at[off + j], acc.at[cidx.at[b * K + j]],
                                 sem_s, add=True)

        def drain_s(off):
            for j in range(K):
                pltpu.make_async_copy(bufs.at[off + j], acc.at[cidx.at[0]],
                                      sem_s).wait()

        fire_g(0, 0)
        fire_g(K, 1)

        def body(b, _):
            par = (b % 2) * K
            drain_g(par)
            fire_s(par, b)
            drain_s(par)
            fire_g(par, b + 2)
            return 0

        lax.fori_loop(0, NBATCH - 2, body, 0)
        for b in (NBATCH - 2, NBATCH - 1):
            par = (b % 2) * K
            drain_g(par)
            fire_s(par, b)
            drain_s(par)
        for e in range(NBATCH * K, NCHUNK):
            pltpu.async_copy(tbl.at[ridx.at[e]], bufs.at[0], sem_g).wait()
            pltpu.sync_copy(bufs.at[0], acc.at[cidx.at[e]], add=True)

    @pl.when(c == 0)
    def _():
        run(ga_hbm)

    @pl.when(c == 1)
    def _():
        run(gb_hbm)

    plsc.subcore_barrier()
    pltpu.sync_copy(acc.at[pl.ds(s * RPT, RPT)],
                    out_hbm.at[pl.ds(c * NPAD + s * RPT, RPT)])


def _mm1_body(x_ref, w_ref, b_ref, dp_ref, ga_ref, gb_ref, dis_ref):
    h = lax.dot_general(x_ref[...], w_ref[...], (((1,), (1,)), ((), ())),
                        preferred_element_type=jnp.float32) + b_ref[...]
    deg = 1.0 + dp_ref[:, 0:1] + dp_ref[:, 1:2]
    dis = lax.rsqrt(deg)
    dis_ref[...] = dis
    g = dis * h
    ga_ref[...] = g[:, :HD]
    gb_ref[...] = g[:, HD:]


_mm1 = pl.pallas_call(
    _mm1_body,
    grid=(GRID,),
    in_specs=[
        pl.BlockSpec((BLK, D), lambda i: (i, 0)),
        pl.BlockSpec((D, D), lambda i: (0, 0)),
        pl.BlockSpec((1, D), lambda i: (0, 0)),
        pl.BlockSpec((BLK, 2), lambda i: (i, 0)),
    ],
    out_specs=[
        pl.BlockSpec((BLK, HD), lambda i: (i, 0)),
        pl.BlockSpec((BLK, HD), lambda i: (i, 0)),
        pl.BlockSpec((BLK, 1), lambda i: (i, 0)),
    ],
    out_shape=[
        jax.ShapeDtypeStruct((N, HD), jnp.float32),
        jax.ShapeDtypeStruct((N, HD), jnp.float32),
        jax.ShapeDtypeStruct((N, 1), jnp.float32),
    ],
)


def _mm2_body(ga_ref, gb_ref, pa_ref, pb_ref, dis_ref, w_ref, b_ref,
              g2a_ref, g2b_ref):
    dis = dis_ref[...]
    ha = jnp.maximum(dis * (ga_ref[...] + pa_ref[0]), 0.0)
    hb = jnp.maximum(dis * (gb_ref[...] + pb_ref[0]), 0.0)
    mm = lax.dot_general(ha, w_ref[:, :HD], (((1,), (1,)), ((), ())),
                         preferred_element_type=jnp.float32)
    mm = mm + lax.dot_general(hb, w_ref[:, HD:], (((1,), (1,)), ((), ())),
                              preferred_element_type=jnp.float32)
    g2 = dis * (mm + b_ref[...])
    g2a_ref[...] = g2[:, :HD]
    g2b_ref[...] = g2[:, HD:]


_mm2 = pl.pallas_call(
    _mm2_body,
    grid=(GRID,),
    in_specs=[
        pl.BlockSpec((BLK, HD), lambda i: (i, 0)),
        pl.BlockSpec((BLK, HD), lambda i: (i, 0)),
        pl.BlockSpec((1, BLK, HD), lambda i: (0, i, 0)),
        pl.BlockSpec((1, BLK, HD), lambda i: (1, i, 0)),
        pl.BlockSpec((BLK, 1), lambda i: (i, 0)),
        pl.BlockSpec((D, D), lambda i: (0, 0)),
        pl.BlockSpec((1, D), lambda i: (0, 0)),
    ],
    out_specs=[
        pl.BlockSpec((BLK, HD), lambda i: (i, 0)),
        pl.BlockSpec((BLK, HD), lambda i: (i, 0)),
    ],
    out_shape=[
        jax.ShapeDtypeStruct((N, HD), jnp.float32),
        jax.ShapeDtypeStruct((N, HD), jnp.float32),
    ],
)


def _fin_body(g2a_ref, g2b_ref, pa_ref, pb_ref, dis_ref, o_ref):
    dis = dis_ref[...]
    o_ref[:, :HD] = dis * (g2a_ref[...] + pa_ref[0])
    o_ref[:, HD:] = dis * (g2b_ref[...] + pb_ref[0])


_fin = pl.pallas_call(
    _fin_body,
    grid=(GRID,),
    in_specs=[
        pl.BlockSpec((BLK, HD), lambda i: (i, 0)),
        pl.BlockSpec((BLK, HD), lambda i: (i, 0)),
        pl.BlockSpec((1, BLK, HD), lambda i: (0, i, 0)),
        pl.BlockSpec((1, BLK, HD), lambda i: (1, i, 0)),
        pl.BlockSpec((BLK, 1), lambda i: (i, 0)),
    ],
    out_specs=pl.BlockSpec((BLK, D), lambda i: (i, 0)),
    out_shape=jax.ShapeDtypeStruct((N, D), jnp.float32),
)


def kernel(x, edge_index, W1, b1, W2, b2):
    ei = edge_index.astype(jnp.int32)
    rowd = ei[0].reshape(NS, NCHUNK, CH)
    cold = ei[1].reshape(NS, NCHUNK, CH)
    colw = ei[1].reshape(NC * NS, DCH, CH)
    degp = _deg_kernel(colw)
    degT = jnp.transpose(degp.reshape(NC, NPAD))
    b1r = b1.reshape(1, D)
    b2r = b2.reshape(1, D)
    g1a, g1b, dis = _mm1(x, W1, b1r, degT)
    p1 = _scatter_kernel(g1a, g1b, rowd, cold).reshape(NC, NPAD, HD)
    g2a, g2b = _mm2(g1a, g1b, p1, p1, dis, W2, b2r)
    p2 = _scatter_kernel(g2a, g2b, rowd, cold).reshape(NC, NPAD, HD)
    return _fin(g2a, g2b, p2, p2, dis)


# speedup vs baseline: 2.0508x; 1.0634x over previous
import functools

import jax
import jax.numpy as jnp
from jax import lax
from jax.experimental import pallas as pl
from jax.experimental.pallas import tpu as pltpu
from jax.experimental.pallas import tpu_sc as plsc

N = 10000
E = 320000
D = 128
HD = D // 2
NC, NS = 2, 16
EPT = E // NS
CH = 80
NCHUNK = EPT // CH
K = 4
NBATCH = NCHUNK // K
NPAD = 10240
RPT = NPAD // NS
ZR = 80
EPW = E // (NC * NS)
DCH = EPW // CH
BLK = 2000
GRID = N // BLK

_sc_mesh = plsc.VectorSubcoreMesh(core_axis_name="c", subcore_axis_name="s")


@functools.partial(
    pl.kernel,
    out_type=jax.ShapeDtypeStruct((NC, NPAD), jnp.float32),
    mesh=_sc_mesh,
    scratch_types=[
        pltpu.VMEM((DCH, CH), jnp.int32),
        pltpu.VMEM((CH,), jnp.float32),
        pltpu.VMEM((RPT,), jnp.float32),
        pltpu.VMEM_SHARED((NPAD,), jnp.float32),
        pltpu.SemaphoreType.DMA,
    ],
    compiler_params=pltpu.CompilerParams(use_tc_tiling_on_sc=False),
)
def _deg_kernel(eir_hbm, out_hbm, cidx, ones_v, zer_v, acc, sem):
    c = lax.axis_index("c")
    s = lax.axis_index("s")
    for j in range(CH // 16):
        ones_v[pl.ds(j * 16, 16)] = jnp.ones((16,), jnp.float32)

    def zbody(j, _):
        zer_v[pl.ds(j * 16, 16)] = jnp.zeros((16,), jnp.float32)
        return 0

    lax.fori_loop(0, RPT // 16, zbody, 0)
    pltpu.sync_copy(zer_v, acc.at[pl.ds(s * RPT, RPT)])
    pltpu.sync_copy(eir_hbm.at[1, s, pl.ds(c * DCH, DCH)], cidx)
    plsc.subcore_barrier()

    def body(i, _):
        pltpu.async_copy(ones_v, acc.at[cidx.at[i]], sem, add=True)
        return 0

    lax.fori_loop(0, DCH, body, 0)

    def drain(i, _):
        pltpu.make_async_copy(ones_v, acc.at[cidx.at[0]], sem).wait()
        return 0

    lax.fori_loop(0, DCH, drain, 0)
    plsc.subcore_barrier()
    pltpu.sync_copy(acc.at[pl.ds(s * RPT, RPT)],
                    out_hbm.at[c, pl.ds(s * RPT, RPT)])


@functools.partial(
    pl.kernel,
    out_type=jax.ShapeDtypeStruct((NC, NPAD, HD), jnp.float32),
    mesh=_sc_mesh,
    scratch_types=[
        pltpu.VMEM((NCHUNK, CH), jnp.int32),
        pltpu.VMEM((NCHUNK, CH), jnp.int32),
        pltpu.VMEM((2 * K, CH, HD), jnp.float32),
        pltpu.VMEM((ZR, HD), jnp.float32),
        pltpu.VMEM_SHARED((NPAD, HD), jnp.float32),
        pltpu.SemaphoreType.DMA,
        pltpu.SemaphoreType.DMA,
    ],
    compiler_params=pltpu.CompilerParams(use_tc_tiling_on_sc=False),
)
def _scatter_kernel(ga_hbm, gb_hbm, eir_hbm, out_hbm, ridx, cidx,
                    bufs, zer_v, acc, sem_g, sem_s):
    c = lax.axis_index("c")
    s = lax.axis_index("s")

    def zfill(i, _):
        for j in range(HD // 16):
            zer_v[i, pl.ds(j * 16, 16)] = jnp.zeros((16,), jnp.float32)
        return 0

    lax.fori_loop(0, ZR, zfill, 0)

    def zcopy(k, _):
        pltpu.sync_copy(zer_v, acc.at[pl.ds(s * RPT + k * ZR, ZR)])
        return 0

    lax.fori_loop(0, RPT // ZR, zcopy, 0)
    pltpu.sync_copy(eir_hbm.at[0, s], ridx)
    pltpu.sync_copy(eir_hbm.at[1, s], cidx)
    plsc.subcore_barrier()

    def run(tbl):
        def fire_g(off, b):
            for j in range(K):
                pltpu.async_copy(tbl.at[ridx.at[b * K + j]], bufs.at[off + j],
                                 sem_g)

        def drain_g(off):
            for j in range(K):
                pltpu.make_async_copy(tbl.at[ridx.at[0]], bufs.at[off + j],
                                      sem_g).wait()

        def fire_s(off, b):
            for j in range(K):
                pltpu.async_copy(bufs.at[off + j], acc.at[cidx.at[b * K + j]],
                                 sem_s, add=True)

        def drain_s(off):
            for j in range(K):
                pltpu.make_async_copy(bufs.at[off + j], acc.at[cidx.at[0]],
                                      sem_s).wait()

        fire_g(0, 0)
        fire_g(K, 1)

        def body(b, _):
            par = (b % 2) * K
            drain_g(par)
            fire_s(par, b)
            drain_s(par)
            fire_g(par, b + 2)
            return 0

        lax.fori_loop(0, NBATCH - 2, body, 0)
        for b in (NBATCH - 2, NBATCH - 1):
            par = (b % 2) * K
            drain_g(par)
            fire_s(par, b)
            drain_s(par)
        for e in range(NBATCH * K, NCHUNK):
            pltpu.async_copy(tbl.at[ridx.at[e]], bufs.at[0], sem_g).wait()
            pltpu.sync_copy(bufs.at[0], acc.at[cidx.at[e]], add=True)

    @pl.when(c == 0)
    def _():
        run(ga_hbm)

    @pl.when(c == 1)
    def _():
        run(gb_hbm)

    plsc.subcore_barrier()
    pltpu.sync_copy(acc.at[pl.ds(s * RPT, RPT)],
                    out_hbm.at[c, pl.ds(s * RPT, RPT)])


def _mm1_body(x_ref, w_ref, b_ref, dp_ref, ga_ref, gb_ref, dis_ref):
    h = lax.dot_general(x_ref[...], w_ref[...], (((1,), (1,)), ((), ())),
                        preferred_element_type=jnp.float32) + b_ref[...]
    deg = 1.0 + dp_ref[:, 0:1] + dp_ref[:, 1:2]
    dis = lax.rsqrt(deg)
    dis_ref[...] = dis
    g = dis * h
    ga_ref[...] = g[:, :HD]
    gb_ref[...] = g[:, HD:]


_mm1 = pl.pallas_call(
    _mm1_body,
    grid=(GRID,),
    in_specs=[
        pl.BlockSpec((BLK, D), lambda i: (i, 0)),
        pl.BlockSpec((D, D), lambda i: (0, 0)),
        pl.BlockSpec((1, D), lambda i: (0, 0)),
        pl.BlockSpec((BLK, 2), lambda i: (i, 0)),
    ],
    out_specs=[
        pl.BlockSpec((BLK, HD), lambda i: (i, 0)),
        pl.BlockSpec((BLK, HD), lambda i: (i, 0)),
        pl.BlockSpec((BLK, 1), lambda i: (i, 0)),
    ],
    out_shape=[
        jax.ShapeDtypeStruct((N, HD), jnp.float32),
        jax.ShapeDtypeStruct((N, HD), jnp.float32),
        jax.ShapeDtypeStruct((N, 1), jnp.float32),
    ],
)


def _mm2_body(ga_ref, gb_ref, pa_ref, pb_ref, dis_ref, w_ref, b_ref,
              g2a_ref, g2b_ref):
    dis = dis_ref[...]
    ha = jnp.maximum(dis * (ga_ref[...] + pa_ref[0]), 0.0)
    hb = jnp.maximum(dis * (gb_ref[...] + pb_ref[0]), 0.0)
    mm = lax.dot_general(ha, w_ref[:, :HD], (((1,), (1,)), ((), ())),
                         preferred_element_type=jnp.float32)
    mm = mm + lax.dot_general(hb, w_ref[:, HD:], (((1,), (1,)), ((), ())),
                              preferred_element_type=jnp.float32)
    g2 = dis * (mm + b_ref[...])
    g2a_ref[...] = g2[:, :HD]
    g2b_ref[...] = g2[:, HD:]


_mm2 = pl.pallas_call(
    _mm2_body,
    grid=(GRID,),
    in_specs=[
        pl.BlockSpec((BLK, HD), lambda i: (i, 0)),
        pl.BlockSpec((BLK, HD), lambda i: (i, 0)),
        pl.BlockSpec((1, BLK, HD), lambda i: (0, i, 0)),
        pl.BlockSpec((1, BLK, HD), lambda i: (1, i, 0)),
        pl.BlockSpec((BLK, 1), lambda i: (i, 0)),
        pl.BlockSpec((D, D), lambda i: (0, 0)),
        pl.BlockSpec((1, D), lambda i: (0, 0)),
    ],
    out_specs=[
        pl.BlockSpec((BLK, HD), lambda i: (i, 0)),
        pl.BlockSpec((BLK, HD), lambda i: (i, 0)),
    ],
    out_shape=[
        jax.ShapeDtypeStruct((N, HD), jnp.float32),
        jax.ShapeDtypeStruct((N, HD), jnp.float32),
    ],
)


def _fin_body(g2a_ref, g2b_ref, pa_ref, pb_ref, dis_ref, o_ref):
    dis = dis_ref[...]
    o_ref[:, :HD] = dis * (g2a_ref[...] + pa_ref[0])
    o_ref[:, HD:] = dis * (g2b_ref[...] + pb_ref[0])


_fin = pl.pallas_call(
    _fin_body,
    grid=(GRID,),
    in_specs=[
        pl.BlockSpec((BLK, HD), lambda i: (i, 0)),
        pl.BlockSpec((BLK, HD), lambda i: (i, 0)),
        pl.BlockSpec((1, BLK, HD), lambda i: (0, i, 0)),
        pl.BlockSpec((1, BLK, HD), lambda i: (1, i, 0)),
        pl.BlockSpec((BLK, 1), lambda i: (i, 0)),
    ],
    out_specs=pl.BlockSpec((BLK, D), lambda i: (i, 0)),
    out_shape=jax.ShapeDtypeStruct((N, D), jnp.float32),
)


def kernel(x, edge_index, W1, b1, W2, b2):
    eir = edge_index.astype(jnp.int32).reshape(2, NS, NCHUNK, CH)
    degp = _deg_kernel(eir)
    degT = jnp.transpose(degp)
    b1r = b1.reshape(1, D)
    b2r = b2.reshape(1, D)
    g1a, g1b, dis = _mm1(x, W1, b1r, degT)
    p1 = _scatter_kernel(g1a, g1b, eir)
    g2a, g2b = _mm2(g1a, g1b, p1, p1, dis, W2, b2r)
    p2 = _scatter_kernel(g2a, g2b, eir)
    return _fin(g2a, g2b, p2, p2, dis)
